# baseline (speedup 1.0000x reference)
import jax
import jax.numpy as jnp
import numpy as np
from jax.experimental import pallas as pl
from jax.experimental.pallas import tpu as pltpu

B, H, S, D = 8, 16, 4096, 128
SUB = 64
N_CHUNKS = S // SUB
N_SEL = 2048 // SUB
SCALE = 1.0 / np.sqrt(D)


def _score_kernel(q_ref, k_ref, ts_ref, sel_ref):
    q = q_ref[0, 0, 0, :]
    k = k_ref[0, 0, :, :]
    qb = q.astype(jnp.bfloat16).astype(jnp.float32)
    kb = k.astype(jnp.bfloat16).astype(jnp.float32)
    t = jnp.sum(kb * qb[None, :], axis=1)
    ts_ref[0, 0, :, :] = t.reshape(N_CHUNKS, SUB) * SCALE

    kc = k.reshape(N_CHUNKS, SUB, D)
    kmaxb = kc.max(axis=1).astype(jnp.bfloat16).astype(jnp.float32)
    kminb = kc.min(axis=1).astype(jnp.bfloat16).astype(jnp.float32)
    s_max = jnp.sum(kmaxb * qb[None, :], axis=1)
    s_min = jnp.sum(kminb * qb[None, :], axis=1)
    cs = jnp.maximum(s_max, s_min)

    ci = cs[:, None]
    cj = cs[None, :]
    ii = jax.lax.broadcasted_iota(jnp.int32, (N_CHUNKS, N_CHUNKS), 0)
    jj = jax.lax.broadcasted_iota(jnp.int32, (N_CHUNKS, N_CHUNKS), 1)
    beats = (cj > ci) | ((cj == ci) & (jj < ii))
    rank = jnp.sum(beats.astype(jnp.float32), axis=1)
    maskf = (rank < float(N_SEL)).astype(jnp.float32)

    tri = (jj <= ii).astype(jnp.float32)
    pos = jnp.sum(tri * maskf[None, :], axis=1) - 1.0
    rr = jax.lax.broadcasted_iota(jnp.int32, (N_CHUNKS, N_SEL), 1).astype(jnp.float32)
    onehot = (pos[:, None] == rr) * maskf[:, None]
    idxf = jax.lax.broadcasted_iota(jnp.int32, (N_CHUNKS, N_SEL), 0).astype(jnp.float32)
    sel = jnp.sum(onehot * idxf, axis=0)
    sel_ref[0, 0, 0, :] = sel.astype(jnp.int32)


def _attn_kernel(sel_ref, ts_ref, *vrefs_out):
    vrefs = vrefs_out[:N_SEL]
    out_ref = vrefs_out[N_SEL]
    b = pl.program_id(0)
    h = pl.program_id(1)
    rows = [ts_ref[0, 0, sel_ref[b, h, j], :] for j in range(N_SEL)]
    s = jnp.stack(rows, axis=0)
    m = jnp.max(s)
    p = jnp.exp(s - m)
    denom = jnp.sum(p)
    pt = jnp.transpose(p)
    acc = pt[:, 0:1] * vrefs[0][0, 0, :, :]
    for j in range(1, N_SEL):
        acc = acc + pt[:, j:j + 1] * vrefs[j][0, 0, :, :]
    out = jnp.sum(acc, axis=0) / denom
    out_ref[0, 0, 0, :] = out


def _make_v_spec(j):
    return pl.BlockSpec((1, 1, SUB, D), lambda b, h, sel, j=j: (b, h, sel[b, h, j], 0))


@jax.jit
def kernel(q, k_cache, v_cache):
    ts, sel = pl.pallas_call(
        _score_kernel,
        grid=(B, H),
        in_specs=[
            pl.BlockSpec((1, 1, 1, D), lambda b, h: (b, h, 0, 0)),
            pl.BlockSpec((1, 1, S, D), lambda b, h: (b, h, 0, 0)),
        ],
        out_specs=[
            pl.BlockSpec((1, 1, N_CHUNKS, SUB), lambda b, h: (b, h, 0, 0)),
            pl.BlockSpec((1, 1, 1, N_SEL), lambda b, h: (b, h, 0, 0)),
        ],
        out_shape=[
            jax.ShapeDtypeStruct((B, H, N_CHUNKS, SUB), jnp.float32),
            jax.ShapeDtypeStruct((B, H, 1, N_SEL), jnp.int32),
        ],
    )(q.reshape(B, H, 1, D), k_cache)

    grid_spec = pltpu.PrefetchScalarGridSpec(
        num_scalar_prefetch=1,
        grid=(B, H),
        in_specs=[pl.BlockSpec((1, 1, N_CHUNKS, SUB), lambda b, h, sel: (b, h, 0, 0))]
        + [_make_v_spec(j) for j in range(N_SEL)],
        out_specs=pl.BlockSpec((1, 1, 1, D), lambda b, h, sel: (b, h, 0, 0)),
    )
    out = pl.pallas_call(
        _attn_kernel,
        grid_spec=grid_spec,
        out_shape=jax.ShapeDtypeStruct((B, H, 1, D), jnp.float32),
    )(sel.reshape(B, H, N_SEL), ts, *([v_cache] * N_SEL))
    return out.reshape(B, H, D)
